# baseline (speedup 1.0000x reference)
import jax
import jax.numpy as jnp
from jax.experimental import pallas as pl
from jax.experimental.pallas import tpu as pltpu

_B, _SUP, _C, _V, _L = 4, 3, 32, 2048, 12
_K_LIST = (10, 20, 40)
_R = 256
_VSTEPS = _V // _R
_NWAVE = 8
_NCH = _V // 128


def _fused_kernel(a_ref, x_ref, u_ref, u0_ref, beta_ref, o_ref, thr_ref):
    i = pl.program_id(1)
    v = pl.program_id(2)

    a = a_ref[0, 0]
    a3 = a.reshape(_R, _V // 128, 128)

    m = jnp.max(a, axis=1, keepdims=True)
    e = jnp.exp(a - m)
    thr_ref[:, pl.ds(0, 1)] = m - 0.5
    thr_ref[:, pl.ds(1, 1)] = m - 1.0
    thr_ref[:, pl.ds(2, 1)] = m - 1.5

    @pl.when(jnp.logical_and(i == 0, v == 0))
    def _init():
        z0 = jnp.dot(u0_ref[...], x_ref[0].reshape(_C, _L * _V),
                     preferred_element_type=jnp.float32)
        z0 = z0.reshape(_C, _L, _V) + beta_ref[...][:, :, None]
        o_ref[0] = z0.reshape(_C * _L, _V)

    xs = x_ref[0, :, :, pl.ds(v * _R, _R)].reshape(_C, _L * _R)
    acc = o_ref[0]
    for t in range(4):
        g = e
        s = jnp.sum(g, axis=1, keepdims=True) + t
        zt = jnp.dot(u_ref[0, t], xs,
                     preferred_element_type=jnp.float32)
        zt = zt.reshape(_C * _L, _R)
        zs = zt * (1.0 / s).reshape(1, _R)
        acc = acc + jnp.dot(zs.astype(jnp.bfloat16), g.astype(jnp.bfloat16),
                            preferred_element_type=jnp.float32)
    o_ref[0] = acc


def kernel(x, support, W0, b0, W1, b1, W2, b2, W3, b3, Wf, bf):
    C = _C
    Ws = [W1, W2, W3]
    bs = [b1, b2, b3]
    U0 = Wf[:, 0:C] @ W0
    Ust = jnp.stack([
        jnp.stack([Wf[:, C * (i + 1):C * (i + 2)] @ Ws[i][:, C * t:C * (t + 1)]
                   for t in range(4)])
        for i in range(_SUP)
    ])
    beta = bf + Wf[:, 0:C] @ b0
    for i in range(_SUP):
        beta = beta + Wf[:, C * (i + 1):C * (i + 2)] @ bs[i]

    xt = x.transpose(0, 1, 3, 2)

    out = pl.pallas_call(
        _fused_kernel,
        grid=(_B, _SUP, _VSTEPS),
        in_specs=[
            pl.BlockSpec((1, 1, _R, _V), lambda n, i, v: (n, i, v, 0)),
            pl.BlockSpec((1, C, _L, _V), lambda n, i, v: (n, 0, 0, 0)),
            pl.BlockSpec((1, 4, C, C), lambda n, i, v: (i, 0, 0, 0)),
            pl.BlockSpec((C, C), lambda n, i, v: (0, 0)),
            pl.BlockSpec((C, _L), lambda n, i, v: (0, 0)),
        ],
        out_specs=pl.BlockSpec((1, C * _L, _V), lambda n, i, v: (n, 0, 0)),
        out_shape=jax.ShapeDtypeStruct((_B, C * _L, _V), jnp.float32),
        scratch_shapes=[pltpu.VMEM((_R, 128), jnp.float32)],
    )(support, xt, Ust, U0, jnp.broadcast_to(beta[:, None], (C, _L)))
    return out.reshape(_B, C, _L, _V).transpose(0, 1, 3, 2)


# speedup vs baseline: 4.8222x; 4.8222x over previous
import jax
import jax.numpy as jnp
from jax.experimental import pallas as pl

_B, _SUP, _C, _V, _L = 4, 3, 32, 2048, 12
_K_LIST = (10, 20, 40)
_CL = _C * _L
_CLP = _CL + 8
_R = 256
_VSTEPS = _V // _R


def _fused_kernel(a_ref, x_ref, xs_ref, u_ref, u0_ref, o_ref):
    i = pl.program_id(1)
    v = pl.program_id(2)

    a = a_ref[0, 0]
    m = jnp.max(a, axis=1, keepdims=True)
    e = jnp.exp(a - m)

    cur = a
    ts = []
    for j in range(_K_LIST[-1]):
        mx = jnp.max(cur, axis=1, keepdims=True)
        if (j + 1) in _K_LIST:
            ts.append(mx)
        if j + 1 < _K_LIST[-1]:
            cur = jnp.where(cur >= mx, -3.0e38, cur)

    @pl.when(jnp.logical_and(i == 0, v == 0))
    def _init():
        o_ref[0] = jnp.dot(u0_ref[...], x_ref[0],
                           preferred_element_type=jnp.float32)

    xs = xs_ref[0].astype(jnp.bfloat16)
    acc = o_ref[0]
    for t in range(4):
        if t < 3:
            g = jnp.where(a >= ts[t], e, 0.0)
        else:
            g = e
        s = jnp.sum(g, axis=1, keepdims=True)
        zt = jnp.dot(u_ref[0, t].astype(jnp.bfloat16), xs,
                     preferred_element_type=jnp.float32)
        zs = zt * (1.0 / s).reshape(1, _R)
        acc = acc + jnp.dot(zs.astype(jnp.bfloat16), g.astype(jnp.bfloat16),
                            preferred_element_type=jnp.float32)
    o_ref[0] = acc


def _lift(u):
    return jnp.kron(u, jnp.eye(_L, dtype=u.dtype))


def kernel(x, support, W0, b0, W1, b1, W2, b2, W3, b3, Wf, bf):
    C = _C
    Ws = [W1, W2, W3]
    bs = [b1, b2, b3]
    U0 = Wf[:, 0:C] @ W0
    beta = bf + Wf[:, 0:C] @ b0
    for i in range(_SUP):
        beta = beta + Wf[:, C * (i + 1):C * (i + 2)] @ bs[i]

    Ust = jnp.stack([
        jnp.stack([_lift(Wf[:, C * (i + 1):C * (i + 2)]
                         @ Ws[i][:, C * t:C * (t + 1)]) for t in range(4)])
        for i in range(_SUP)
    ])
    beta_col = jnp.broadcast_to(beta[:, None], (C, _L)).reshape(_CL, 1)
    U0L = jnp.concatenate(
        [_lift(U0), beta_col, jnp.zeros((_CL, _CLP - _CL - 1), x.dtype)],
        axis=1)

    xt = x.transpose(0, 1, 3, 2).reshape(_B, _CL, _V)
    xt1 = jnp.concatenate(
        [xt, jnp.ones((_B, 1, _V), xt.dtype),
         jnp.zeros((_B, _CLP - _CL - 1, _V), xt.dtype)], axis=1)

    out = pl.pallas_call(
        _fused_kernel,
        grid=(_B, _SUP, _VSTEPS),
        in_specs=[
            pl.BlockSpec((1, 1, _R, _V), lambda n, i, v: (n, i, v, 0)),
            pl.BlockSpec((1, _CLP, _V), lambda n, i, v: (n, 0, 0)),
            pl.BlockSpec((1, _CL, _R), lambda n, i, v: (n, 0, v)),
            pl.BlockSpec((1, 4, _CL, _CL), lambda n, i, v: (i, 0, 0, 0)),
            pl.BlockSpec((_CL, _CLP), lambda n, i, v: (0, 0)),
        ],
        out_specs=pl.BlockSpec((1, _CL, _V), lambda n, i, v: (n, 0, 0)),
        out_shape=jax.ShapeDtypeStruct((_B, _CL, _V), jnp.float32),
    )(support, xt1, xt, Ust, U0L)
    return out.reshape(_B, C, _L, _V).transpose(0, 1, 3, 2)


# speedup vs baseline: 4.9017x; 1.0165x over previous
import jax
import jax.numpy as jnp
from jax.experimental import pallas as pl

_B, _SUP, _C, _V, _L = 4, 3, 32, 2048, 12
_K_LIST = (10, 20, 40)
_CL = _C * _L
_CLP = _CL + 8
_R = 512
_VSTEPS = _V // _R


def _fused_kernel(a_ref, x_ref, xs_ref, u_ref, u0_ref, o_ref):
    i = pl.program_id(1)
    v = pl.program_id(2)

    a = a_ref[0, 0]
    m = jnp.max(a, axis=1, keepdims=True)
    e = jnp.exp(a - m)

    cur = a
    ts = []
    for j in range(_K_LIST[-1]):
        mx = jnp.max(cur, axis=1, keepdims=True)
        if (j + 1) in _K_LIST:
            ts.append(mx)
        if j + 1 < _K_LIST[-1]:
            cur = jnp.where(cur >= mx, -3.0e38, cur)

    @pl.when(jnp.logical_and(i == 0, v == 0))
    def _init():
        o_ref[0] = jnp.dot(u0_ref[...], x_ref[0],
                           preferred_element_type=jnp.float32)

    xs = xs_ref[0].astype(jnp.bfloat16)
    acc = o_ref[0]
    for t in range(4):
        if t < 3:
            g = jnp.where(a >= ts[t], e, 0.0)
        else:
            g = e
        s = jnp.sum(g, axis=1, keepdims=True)
        zt = jnp.dot(u_ref[0, t].astype(jnp.bfloat16), xs,
                     preferred_element_type=jnp.float32)
        zs = zt * (1.0 / s).reshape(1, _R)
        acc = acc + jnp.dot(zs.astype(jnp.bfloat16), g.astype(jnp.bfloat16),
                            preferred_element_type=jnp.float32)
    o_ref[0] = acc


def _lift(u):
    return jnp.kron(u, jnp.eye(_L, dtype=u.dtype))


def kernel(x, support, W0, b0, W1, b1, W2, b2, W3, b3, Wf, bf):
    C = _C
    Ws = [W1, W2, W3]
    bs = [b1, b2, b3]
    U0 = Wf[:, 0:C] @ W0
    beta = bf + Wf[:, 0:C] @ b0
    for i in range(_SUP):
        beta = beta + Wf[:, C * (i + 1):C * (i + 2)] @ bs[i]

    Ust = jnp.stack([
        jnp.stack([_lift(Wf[:, C * (i + 1):C * (i + 2)]
                         @ Ws[i][:, C * t:C * (t + 1)]) for t in range(4)])
        for i in range(_SUP)
    ])
    beta_col = jnp.broadcast_to(beta[:, None], (C, _L)).reshape(_CL, 1)
    U0L = jnp.concatenate(
        [_lift(U0), beta_col, jnp.zeros((_CL, _CLP - _CL - 1), x.dtype)],
        axis=1)

    xt = x.transpose(0, 1, 3, 2).reshape(_B, _CL, _V)
    xt1 = jnp.concatenate(
        [xt, jnp.ones((_B, 1, _V), xt.dtype),
         jnp.zeros((_B, _CLP - _CL - 1, _V), xt.dtype)], axis=1)

    out = pl.pallas_call(
        _fused_kernel,
        grid=(_B, _SUP, _VSTEPS),
        in_specs=[
            pl.BlockSpec((1, 1, _R, _V), lambda n, i, v: (n, i, v, 0)),
            pl.BlockSpec((1, _CLP, _V), lambda n, i, v: (n, 0, 0)),
            pl.BlockSpec((1, _CL, _R), lambda n, i, v: (n, 0, v)),
            pl.BlockSpec((1, 4, _CL, _CL), lambda n, i, v: (i, 0, 0, 0)),
            pl.BlockSpec((_CL, _CLP), lambda n, i, v: (0, 0)),
        ],
        out_specs=pl.BlockSpec((1, _CL, _V), lambda n, i, v: (n, 0, 0)),
        out_shape=jax.ShapeDtypeStruct((_B, _CL, _V), jnp.float32),
    )(support, xt1, xt, Ust, U0L)
    return out.reshape(_B, C, _L, _V).transpose(0, 1, 3, 2)
